# baseline (speedup 1.0000x reference)
import functools

import jax
import jax.numpy as jnp
from jax import lax
from jax.experimental import pallas as pl
from jax.experimental.pallas import tpu as pltpu
from jax.experimental.pallas import tpu_sc as plsc

_B, _S, _D = 4, 4096, 1024
_NW = 32
_S_PER_W = _S // _NW
_R = 16
_CHUNK = _R * _D
_N_CHUNKS = _S_PER_W // _R

_mesh = plsc.VectorSubcoreMesh(core_axis_name="c", subcore_axis_name="s")


@functools.partial(
    pl.kernel,
    mesh=_mesh,
    out_type=jax.ShapeDtypeStruct((_B * _S * _D,), jnp.float32),
    scratch_types=[
        pltpu.VMEM((_CHUNK,), jnp.float32),
        pltpu.VMEM((_CHUNK,), jnp.float32),
    ],
)
def _sc_add(x_hbm, t_hbm, o_hbm, xbuf, tbuf):
    wid = lax.axis_index("s") * 2 + lax.axis_index("c")
    s0 = wid * _S_PER_W

    def vadd_loop(i, carry):
        sl = pl.ds(i * 16, 16)
        xbuf[sl] = xbuf[sl] + tbuf[sl]
        return carry

    for c in range(_N_CHUNKS):
        t_off = (s0 + c * _R) * _D
        pltpu.sync_copy(t_hbm.at[pl.ds(t_off, _CHUNK)], tbuf)
        for b in range(_B):
            x_off = b * _S * _D + t_off
            pltpu.sync_copy(x_hbm.at[pl.ds(x_off, _CHUNK)], xbuf)
            lax.fori_loop(0, _CHUNK // 16, vadd_loop, 0)
            pltpu.sync_copy(xbuf, o_hbm.at[pl.ds(x_off, _CHUNK)])


def kernel(x, pos_table):
    out = _sc_add(x.reshape(-1), pos_table.reshape(-1))
    return out.reshape(x.shape)


# speedup vs baseline: 1.3942x; 1.3942x over previous
import functools

import jax
import jax.numpy as jnp
from jax import lax
from jax.experimental import pallas as pl
from jax.experimental.pallas import tpu as pltpu
from jax.experimental.pallas import tpu_sc as plsc

_B, _S, _D = 4, 4096, 1024
_NW = 32
_S_PER_W = _S // _NW
_R = 16
_CHUNK = _R * _D
_N_CHUNKS = _S_PER_W // _R

_mesh = plsc.VectorSubcoreMesh(core_axis_name="c", subcore_axis_name="s")


@functools.partial(
    pl.kernel,
    mesh=_mesh,
    out_type=jax.ShapeDtypeStruct((_B * _S * _D,), jnp.float32),
    scratch_types=[
        pltpu.VMEM((_CHUNK,), jnp.float32),
        pltpu.VMEM((_CHUNK,), jnp.float32),
    ],
)
def _sc_add(x_hbm, t_hbm, o_hbm, xbuf, tbuf):
    wid = lax.axis_index("s") * 2 + lax.axis_index("c")
    s0 = wid * _S_PER_W

    def vadd_loop(i, carry):
        base = i * 128
        for u in range(8):
            sl = pl.ds(base + u * 16, 16)
            xbuf[sl] = xbuf[sl] + tbuf[sl]
        return carry

    for c in range(_N_CHUNKS):
        t_off = (s0 + c * _R) * _D
        pltpu.sync_copy(t_hbm.at[pl.ds(t_off, _CHUNK)], tbuf)
        for b in range(_B):
            x_off = b * _S * _D + t_off
            pltpu.sync_copy(x_hbm.at[pl.ds(x_off, _CHUNK)], xbuf)
            lax.fori_loop(0, _CHUNK // 128, vadd_loop, 0)
            pltpu.sync_copy(xbuf, o_hbm.at[pl.ds(x_off, _CHUNK)])


def kernel(x, pos_table):
    out = _sc_add(x.reshape(-1), pos_table.reshape(-1))
    return out.reshape(x.shape)


# speedup vs baseline: 2.7649x; 1.9830x over previous
import functools

import jax
import jax.numpy as jnp
from jax import lax
from jax.experimental import pallas as pl
from jax.experimental.pallas import tpu as pltpu
from jax.experimental.pallas import tpu_sc as plsc

_B, _S, _D = 4, 4096, 1024
_NW = 32
_S_PER_W = _S // _NW
_R = 16
_N_CHUNKS = _S_PER_W // _R

_mesh = plsc.VectorSubcoreMesh(core_axis_name="c", subcore_axis_name="s")


@functools.partial(
    pl.kernel,
    mesh=_mesh,
    out_type=jax.ShapeDtypeStruct((_B * _S, _D), jnp.float32),
    scratch_types=[
        pltpu.VMEM((_R, _D), jnp.float32),
        pltpu.VMEM((_R, _D), jnp.float32),
    ],
)
def _sc_add(x_hbm, t_hbm, o_hbm, xbuf, tbuf):
    wid = lax.axis_index("s") * 2 + lax.axis_index("c")
    s0 = wid * _S_PER_W

    def vadd_loop(i, carry):
        r = lax.shift_right_logical(i, 3)
        cb = lax.mul(lax.bitwise_and(i, 7), 128)
        for u in range(8):
            sl = pl.ds(cb + u * 16, 16)
            xbuf[r, sl] = xbuf[r, sl] + tbuf[r, sl]
        return carry

    for c in range(_N_CHUNKS):
        srow = s0 + c * _R
        pltpu.sync_copy(t_hbm.at[pl.ds(srow, _R)], tbuf)
        for b in range(_B):
            xrow = b * _S + srow
            pltpu.sync_copy(x_hbm.at[pl.ds(xrow, _R)], xbuf)
            lax.fori_loop(0, _R * _D // 128, vadd_loop, 0)
            pltpu.sync_copy(xbuf, o_hbm.at[pl.ds(xrow, _R)])


def kernel(x, pos_table):
    out = _sc_add(x.reshape(_B * _S, _D), pos_table)
    return out.reshape(x.shape)
